# baseline (speedup 1.0000x reference)
import functools

import jax
import jax.numpy as jnp
from jax import lax
from jax.experimental import pallas as pl
from jax.experimental.pallas import tpu as pltpu
from jax.experimental.pallas import tpu_sc as plsc

NN = 100000
NE = 3200000
CN = 256
CE = 2560
N_FULL_NODE = NN // CN
NODE_REM_BASE = N_FULL_NODE * CN
NODE_REM = 256
N_FULL_EDGE = NE // CE
NW = 32
NW_NODE = 5
NW_EDGE = NW - NW_NODE
K_PAIRS = (N_FULL_EDGE // NW_EDGE + 2) // 2 * 2
KN_PAIRS = (N_FULL_NODE // NW_NODE + 2) // 2 * 2

_f32 = jnp.float32
_i32 = jnp.int32


def _cvt_idx(v, hi):
    i = v.astype(_i32)
    return jnp.minimum(jnp.maximum(i, 0), hi)


def _sc_body(xt_hbm, bond_hbm, dist_hbm, at_hbm, ht_hbm, bt_hbm,
             nout_hbm, eout_hbm, at_v, ht_v, bt_v, xv, nov0, nov1,
             bv0, bv1, eov0, eov1,
             sem0, sem1, semi0, semi1, semn0, semn1):
    c = lax.axis_index("c")
    s = lax.axis_index("s")
    wid = s * 2 + c

    pltpu.sync_copy(at_hbm, at_v)
    pltpu.sync_copy(ht_hbm, ht_v)
    pltpu.sync_copy(bt_hbm, bt_v)

    eovs = (eov0, eov1)
    sems = (sem0, sem1)
    bvs = (bv0, bv1)
    semis = (semi0, semi1)
    novs = (nov0, nov1)
    semns = (semn0, semn1)

    def node_compute(base, width, nov):
        pltpu.sync_copy(xt_hbm.at[pl.ds(0, 16), pl.ds(base, width)],
                        xv.at[pl.ds(0, 16), pl.ds(0, width)])

        @pl.loop(0, width // 16)
        def _grp(g):
            j = g * 16
            sidx = _cvt_idx(xv[0, pl.ds(j, 16)], 38)
            hidx = _cvt_idx(xv[1, pl.ds(j, 16)], 7)
            for r0 in (0, 16):
                vals = [plsc.load_gather(at_v, [sidx + 39 * (r0 + r)])
                        for r in range(16)]
                for r in range(16):
                    nov[r0 + r, pl.ds(j, 16)] = vals[r]
            vals = [plsc.load_gather(ht_v, [hidx + 8 * r]) for r in range(16)]
            for r in range(16):
                nov[32 + r, pl.ds(j, 16)] = vals[r]
            vals = [xv[2 + r, pl.ds(j, 16)] for r in range(14)]
            for r in range(14):
                nov[48 + r, pl.ds(j, 16)] = vals[r]

    def node_out_start(b, base):
        pltpu.async_copy(novs[b], nout_hbm.at[pl.ds(0, 64), pl.ds(base, CN)],
                         semns[b])

    def node_out_wait(b):
        pltpu.make_async_copy(novs[b],
                              nout_hbm.at[pl.ds(0, 64), pl.ds(0, CN)],
                              semns[b]).wait()

    def edge_compute(width, bv, eov):
        @pl.loop(0, width // 16, unroll=2)
        def _grp(g):
            j = g * 16
            bidx = jnp.minimum(bv[pl.ds(j, 16)].astype(_i32).astype(jnp.uint32),
                               jnp.uint32(5)).astype(_i32)
            vals = [plsc.load_gather(bt_v, [bidx + 6 * r]) for r in range(16)]
            for r in range(16):
                eov[r, pl.ds(j, 16)] = vals[r]

    def edge_in_start(b, base):
        pltpu.async_copy(bond_hbm.at[pl.ds(base, CE)], bvs[b], semis[b])

    def edge_in_wait(b):
        pltpu.make_async_copy(bond_hbm.at[pl.ds(0, CE)], bvs[b], semis[b]).wait()

    def edge_out_start(b, base):
        pltpu.async_copy(eovs[b],
                         eout_hbm.at[pl.ds(0, 16), pl.ds(base, CE)], sems[b])
        pltpu.async_copy(dist_hbm.at[pl.ds(0, 1), pl.ds(base, CE)],
                         eout_hbm.at[pl.ds(16, 1), pl.ds(base, CE)], sems[b])

    def edge_out_wait(b):
        pltpu.make_async_copy(eovs[b],
                              eout_hbm.at[pl.ds(0, 16), pl.ds(0, CE)],
                              sems[b]).wait()
        pltpu.make_async_copy(dist_hbm.at[pl.ds(0, 1), pl.ds(0, CE)],
                              eout_hbm.at[pl.ds(16, 1), pl.ds(0, CE)],
                              sems[b]).wait()

    @pl.when(wid < NW_NODE)
    def _node_side():
        n_kn = (N_FULL_NODE - 1 - wid) // NW_NODE + 1

        @pl.loop(0, KN_PAIRS, step=2)
        def _node_pair(k2):
            for b in range(2):
                k = k2 + b
                ci = wid + k * NW_NODE

                @pl.when(k < n_kn)
                def _do():
                    @pl.when(k >= 2)
                    def _drain():
                        node_out_wait(b)

                    node_compute(ci * CN, CN, novs[b])
                    node_out_start(b, ci * CN)

        for b in range(2):
            node_out_wait(b)

        @pl.when(wid == 0)
        def _node_rem():
            base = pl.multiple_of(wid * 0 + NODE_REM_BASE, 128)
            node_compute(base, NODE_REM, nov0)
            pltpu.sync_copy(nov0.at[pl.ds(0, 64), pl.ds(0, NODE_REM)],
                            nout_hbm.at[pl.ds(0, 64), pl.ds(base, NODE_REM)])

    @pl.when(wid >= NW_NODE)
    def _edge_side():
        ew = wid - NW_NODE
        n_k = (N_FULL_EDGE - 1 - ew) // NW_EDGE + 1

        edge_in_start(0, ew * CE)

        @pl.loop(0, K_PAIRS, step=2)
        def _edge_pair(k2):
            for b in range(2):
                k = k2 + b
                ci = ew + k * NW_EDGE

                @pl.when(k < n_k)
                def _do():
                    edge_in_wait(b)

                    @pl.when(k + 1 < n_k)
                    def _prefetch():
                        edge_in_start(1 - b, (ci + NW_EDGE) * CE)

                    @pl.when(k >= 2)
                    def _drain():
                        edge_out_wait(b)

                    edge_compute(CE, bvs[b], eovs[b])
                    edge_out_start(b, ci * CE)

        for b in range(2):
            edge_out_wait(b)


_OUT_TYPE = (
    jax.ShapeDtypeStruct((62, NN), _f32),
    jax.ShapeDtypeStruct((17, NE), _f32),
)

_SCRATCH = [
    pltpu.VMEM((39 * 32,), _f32),
    pltpu.VMEM((8 * 16,), _f32),
    pltpu.VMEM((6 * 16,), _f32),
    pltpu.VMEM((16, CN), _f32),
    pltpu.VMEM((64, CN), _f32),
    pltpu.VMEM((64, CN), _f32),
    pltpu.VMEM((CE,), _f32),
    pltpu.VMEM((CE,), _f32),
    pltpu.VMEM((16, CE), _f32),
    pltpu.VMEM((16, CE), _f32),
    pltpu.SemaphoreType.DMA,
    pltpu.SemaphoreType.DMA,
    pltpu.SemaphoreType.DMA,
    pltpu.SemaphoreType.DMA,
    pltpu.SemaphoreType.DMA,
    pltpu.SemaphoreType.DMA,
]

_MESH = plsc.VectorSubcoreMesh(core_axis_name="c", subcore_axis_name="s")

_sc_call = functools.partial(
    pl.kernel,
    out_type=_OUT_TYPE,
    mesh=_MESH,
    scratch_types=_SCRATCH,
    compiler_params=pltpu.CompilerParams(needs_layout_passes=False),
)(_sc_body)


@jax.jit
def kernel(x, edge_attr, atom_table, hybrid_table, bond_table):
    node_t, edge_t = _sc_call(
        x.T, edge_attr[:, 0], edge_attr[:, 1:2].T, atom_table.T.reshape(-1),
        hybrid_table.T.reshape(-1), bond_table.T.reshape(-1))
    return node_t.T, edge_t.T


# speedup vs baseline: 1.0076x; 1.0076x over previous
import functools

import jax
import jax.numpy as jnp
from jax import lax
from jax.experimental import pallas as pl
from jax.experimental.pallas import tpu as pltpu
from jax.experimental.pallas import tpu_sc as plsc

NN = 100000
NE = 3200000
CN = 256
CE = 2560
N_FULL_NODE = NN // CN
NODE_REM_BASE = N_FULL_NODE * CN
NODE_REM = 256
N_FULL_EDGE = NE // CE
NW = 32
NW_NODE = 5
NW_EDGE = NW - NW_NODE
K_PAIRS = (N_FULL_EDGE // NW_EDGE + 2) // 2 * 2
KN_PAIRS = (N_FULL_NODE // NW_NODE + 2) // 2 * 2

_f32 = jnp.float32
_i32 = jnp.int32


def _cvt_idx(v, hi):
    i = v.astype(_i32)
    return jnp.minimum(jnp.maximum(i, 0), hi)


def _sc_body(xt_hbm, bond_hbm, dist_hbm, at_hbm, ht_hbm, bt_hbm,
             nout_hbm, eout_hbm, at_v, ht_v, bt_v, xv, nov0, nov1,
             bv0, bv1, eov0, eov1,
             sem0, sem1, semi0, semi1, semn0, semn1, semd):
    c = lax.axis_index("c")
    s = lax.axis_index("s")
    wid = s * 2 + c

    pltpu.sync_copy(at_hbm, at_v)
    pltpu.sync_copy(ht_hbm, ht_v)
    pltpu.sync_copy(bt_hbm, bt_v)

    eovs = (eov0, eov1)
    sems = (sem0, sem1)
    bvs = (bv0, bv1)
    semis = (semi0, semi1)
    novs = (nov0, nov1)
    semns = (semn0, semn1)

    def node_compute(base, width, nov):
        pltpu.sync_copy(xt_hbm.at[pl.ds(0, 16), pl.ds(base, width)],
                        xv.at[pl.ds(0, 16), pl.ds(0, width)])

        @pl.loop(0, width // 16)
        def _grp(g):
            j = g * 16
            sidx = _cvt_idx(xv[0, pl.ds(j, 16)], 38)
            hidx = _cvt_idx(xv[1, pl.ds(j, 16)], 7)
            for r0 in (0, 16):
                vals = [plsc.load_gather(at_v, [sidx + 39 * (r0 + r)])
                        for r in range(16)]
                for r in range(16):
                    nov[r0 + r, pl.ds(j, 16)] = vals[r]
            vals = [plsc.load_gather(ht_v, [hidx + 8 * r]) for r in range(16)]
            for r in range(16):
                nov[32 + r, pl.ds(j, 16)] = vals[r]
            vals = [xv[2 + r, pl.ds(j, 16)] for r in range(14)]
            for r in range(14):
                nov[48 + r, pl.ds(j, 16)] = vals[r]

    def node_out_start(b, base):
        pltpu.async_copy(novs[b], nout_hbm.at[pl.ds(0, 64), pl.ds(base, CN)],
                         semns[b])

    def node_out_wait(b):
        pltpu.make_async_copy(novs[b],
                              nout_hbm.at[pl.ds(0, 64), pl.ds(0, CN)],
                              semns[b]).wait()

    def edge_compute(width, bv, eov):
        @pl.loop(0, width // 16, unroll=2)
        def _grp(g):
            j = g * 16
            bidx = jnp.minimum(bv[pl.ds(j, 16)].astype(_i32).astype(jnp.uint32),
                               jnp.uint32(5)).astype(_i32)
            vals = [plsc.load_gather(bt_v, [bidx + 6 * r]) for r in range(16)]
            for r in range(16):
                eov[r, pl.ds(j, 16)] = vals[r]

    def edge_in_start(b, base):
        pltpu.async_copy(bond_hbm.at[pl.ds(base, CE)], bvs[b], semis[b])

    def edge_in_wait(b):
        pltpu.make_async_copy(bond_hbm.at[pl.ds(0, CE)], bvs[b], semis[b]).wait()

    def edge_out_start(b, base):
        pltpu.async_copy(eovs[b],
                         eout_hbm.at[pl.ds(0, 16), pl.ds(base, CE)], sems[b])

    def edge_out_wait(b):
        pltpu.make_async_copy(eovs[b],
                              eout_hbm.at[pl.ds(0, 16), pl.ds(0, CE)],
                              sems[b]).wait()

    @pl.when(wid < NW_NODE)
    def _node_side():
        n_kn = (N_FULL_NODE - 1 - wid) // NW_NODE + 1

        @pl.loop(0, KN_PAIRS, step=2)
        def _node_pair(k2):
            for b in range(2):
                k = k2 + b
                ci = wid + k * NW_NODE

                @pl.when(k < n_kn)
                def _do():
                    @pl.when(k >= 2)
                    def _drain():
                        node_out_wait(b)

                    node_compute(ci * CN, CN, novs[b])
                    node_out_start(b, ci * CN)

        for b in range(2):
            node_out_wait(b)

        @pl.when(wid == 0)
        def _node_rem():
            base = pl.multiple_of(wid * 0 + NODE_REM_BASE, 128)
            node_compute(base, NODE_REM, nov0)
            pltpu.sync_copy(nov0.at[pl.ds(0, 64), pl.ds(0, NODE_REM)],
                            nout_hbm.at[pl.ds(0, 64), pl.ds(base, NODE_REM)])

    @pl.when(wid >= NW_NODE)
    def _edge_side():
        ew = wid - NW_NODE
        n_k = (N_FULL_EDGE - 1 - ew) // NW_EDGE + 1

        @pl.when(ew == 0)
        def _dist_start():
            pltpu.async_copy(dist_hbm,
                             eout_hbm.at[pl.ds(16, 1), pl.ds(0, NE)], semd)

        edge_in_start(0, ew * CE)

        @pl.loop(0, K_PAIRS, step=2)
        def _edge_pair(k2):
            for b in range(2):
                k = k2 + b
                ci = ew + k * NW_EDGE

                @pl.when(k < n_k)
                def _do():
                    edge_in_wait(b)

                    @pl.when(k + 1 < n_k)
                    def _prefetch():
                        edge_in_start(1 - b, (ci + NW_EDGE) * CE)

                    @pl.when(k >= 2)
                    def _drain():
                        edge_out_wait(b)

                    edge_compute(CE, bvs[b], eovs[b])
                    edge_out_start(b, ci * CE)

        for b in range(2):
            edge_out_wait(b)

        @pl.when(ew == 0)
        def _dist_wait():
            pltpu.make_async_copy(dist_hbm,
                                  eout_hbm.at[pl.ds(16, 1), pl.ds(0, NE)],
                                  semd).wait()


_OUT_TYPE = (
    jax.ShapeDtypeStruct((62, NN), _f32),
    jax.ShapeDtypeStruct((17, NE), _f32),
)

_SCRATCH = [
    pltpu.VMEM((39 * 32,), _f32),
    pltpu.VMEM((8 * 16,), _f32),
    pltpu.VMEM((6 * 16,), _f32),
    pltpu.VMEM((16, CN), _f32),
    pltpu.VMEM((64, CN), _f32),
    pltpu.VMEM((64, CN), _f32),
    pltpu.VMEM((CE,), _f32),
    pltpu.VMEM((CE,), _f32),
    pltpu.VMEM((16, CE), _f32),
    pltpu.VMEM((16, CE), _f32),
    pltpu.SemaphoreType.DMA,
    pltpu.SemaphoreType.DMA,
    pltpu.SemaphoreType.DMA,
    pltpu.SemaphoreType.DMA,
    pltpu.SemaphoreType.DMA,
    pltpu.SemaphoreType.DMA,
    pltpu.SemaphoreType.DMA,
]

_MESH = plsc.VectorSubcoreMesh(core_axis_name="c", subcore_axis_name="s")

_sc_call = functools.partial(
    pl.kernel,
    out_type=_OUT_TYPE,
    mesh=_MESH,
    scratch_types=_SCRATCH,
    compiler_params=pltpu.CompilerParams(needs_layout_passes=False),
)(_sc_body)


@jax.jit
def kernel(x, edge_attr, atom_table, hybrid_table, bond_table):
    node_t, edge_t = _sc_call(
        x.T, edge_attr[:, 0], edge_attr[:, 1:2].T, atom_table.T.reshape(-1),
        hybrid_table.T.reshape(-1), bond_table.T.reshape(-1))
    return node_t.T, edge_t.T
